# baseline (speedup 1.0000x reference)
import jax
import jax.numpy as jnp
from jax import lax
from jax.experimental import pallas as pl
from jax.experimental.pallas import tpu as pltpu
from jax.experimental.pallas import tpu_sc as plsc

_B, _S, _D = 16384, 200, 64
_N = _B * _S
_W = 128
_NR = _N // _W

_INFO = plsc.get_sparse_core_info()
_NC, _NS = _INFO.num_cores, _INFO.num_subcores
_NW = _NC * _NS
_RCH = 8
_T = _RCH * _W
_H = _T // 2
_ROWS_W = _NR // _NW
_PER_W = _ROWS_W * _W
_CHUNKS = _ROWS_W // _RCH


def _sc_body(idx_hbm, tab_hbm, out_hbm,
             tab_v, idx_v0, idx_v1, out_v0, out_v1,
             wsem0, wsem1, isem0, isem1):
    wid = lax.axis_index("s") * _NC + lax.axis_index("c")
    row_base = wid * _ROWS_W
    tok_base = wid * _PER_W
    idx_v = (idx_v0, idx_v1)
    out_v = (out_v0, out_v1)
    wsem = (wsem0, wsem1)
    isem = (isem0, isem1)

    def fire_idx(i, b):
        pltpu.async_copy(idx_hbm.at[pl.ds(row_base + i * _RCH, _RCH), :],
                         idx_v[b], isem[b])

    pltpu.sync_copy(tab_hbm, tab_v)
    iota = jnp.arange(16, dtype=jnp.int32)
    cols = [iota + 16 * k for k in range(_D // 16)]
    lanes = [jnp.full((16,), l, jnp.int32) for l in range(16)]
    dnums = lax.GatherDimensionNumbers(
        offset_dims=(), collapsed_slice_dims=(0,), start_index_map=(0,))

    def half(ib, ibuf, tvec0, tok0):
        def group(gi, tvec):
            obase = pl.multiple_of(gi * (16 * _D), 8)
            idxg = plsc.load_gather(idx_v[ibuf], [tvec >> 7, tvec & (_W - 1)])
            for l in range(16):
                splat = lax.gather(
                    idxg, lanes[l][:, None], dnums, (1,),
                    mode=lax.GatherScatterMode.PROMISE_IN_BOUNDS)
                flat0 = splat << 6
                for k in range(_D // 16):
                    val = plsc.load_gather(tab_v, [flat0 + cols[k]])
                    out_v[ib][pl.ds(obase + l * _D + 16 * k, 16)] = val
            return tvec + 16

        lax.fori_loop(0, _H // 16, group, tvec0)
        pltpu.async_copy(out_v[ib], out_hbm.at[pl.ds(tok0 * _D, _H * _D)],
                         wsem[ib])

    fire_idx(0, 0)

    def pair(g, _):
        for b in range(2):
            i = 2 * g + b
            tok0 = tok_base + i * _T
            pltpu.make_async_copy(
                idx_hbm.at[pl.ds(row_base + i * _RCH, _RCH), :],
                idx_v[b], isem[b]).wait()

            @pl.when(i + 1 < _CHUNKS)
            def _():
                fire_idx(i + 1, 1 - b)

            for hb in range(2):
                @pl.when(i >= 1)
                def _():
                    pltpu.make_async_copy(
                        out_v[hb], out_hbm.at[pl.ds(tok0 * _D, _H * _D)],
                        wsem[hb]).wait()
                half(hb, b, iota + hb * _H, tok0 + hb * _H)
        return ()

    lax.fori_loop(0, _CHUNKS // 2, pair, ())
    for b in range(2):
        pltpu.make_async_copy(out_v[b],
                              out_hbm.at[pl.ds(tok_base * _D, _H * _D)],
                              wsem[b]).wait()


def kernel(inputs, table):
    idx = inputs.astype(jnp.int32).reshape(_NR, _W)
    out = pl.kernel(
        _sc_body,
        out_type=jax.ShapeDtypeStruct((_N * _D,), jnp.float32),
        mesh=plsc.VectorSubcoreMesh(core_axis_name="c", subcore_axis_name="s"),
        compiler_params=pltpu.CompilerParams(needs_layout_passes=False),
        scratch_types=[
            pltpu.VMEM((7 * _D,), jnp.float32),
            pltpu.VMEM((_RCH, _W), jnp.int32),
            pltpu.VMEM((_RCH, _W), jnp.int32),
            pltpu.VMEM((_H * _D,), jnp.float32),
            pltpu.VMEM((_H * _D,), jnp.float32),
            pltpu.SemaphoreType.DMA,
            pltpu.SemaphoreType.DMA,
            pltpu.SemaphoreType.DMA,
            pltpu.SemaphoreType.DMA,
        ],
    )(idx, table.reshape(7 * _D))
    return out.reshape(_B, _S, _D)


# speedup vs baseline: 1.0020x; 1.0020x over previous
import jax
import jax.numpy as jnp
from jax import lax
from jax.experimental import pallas as pl
from jax.experimental.pallas import tpu as pltpu
from jax.experimental.pallas import tpu_sc as plsc

_B, _S, _D = 16384, 200, 64
_N = _B * _S
_W = 128
_NR = _N // _W

_INFO = plsc.get_sparse_core_info()
_NC, _NS = _INFO.num_cores, _INFO.num_subcores
_NW = _NC * _NS
_RCH = 8
_T = _RCH * _W
_H = _T // 2
_ROWS_W = _NR // _NW
_PER_W = _ROWS_W * _W
_CHUNKS = _ROWS_W // _RCH


def _sc_body(idx_hbm, tab_hbm, out_hbm,
             tab_v, idx_v0, idx_v1, out_v0, out_v1,
             wsem0, wsem1, isem0, isem1):
    wid = lax.axis_index("s") * _NC + lax.axis_index("c")
    row_base = wid * _ROWS_W
    tok_base = wid * _PER_W
    idx_v = (idx_v0, idx_v1)
    out_v = (out_v0, out_v1)
    wsem = (wsem0, wsem1)
    isem = (isem0, isem1)

    def fire_idx(i, b):
        pltpu.async_copy(idx_hbm.at[pl.ds(row_base + i * _RCH, _RCH), :],
                         idx_v[b], isem[b])

    pltpu.sync_copy(tab_hbm, tab_v)
    iota = jnp.arange(16, dtype=jnp.int32)
    cols = [iota + 16 * k for k in range(_D // 16)]
    lanes = [jnp.full((16,), l, jnp.int32) for l in range(16)]
    dnums = lax.GatherDimensionNumbers(
        offset_dims=(), collapsed_slice_dims=(0,), start_index_map=(0,))

    def half(ib, ibuf, tvec0, tok0):
        def group(gi, tvec):
            obase = pl.multiple_of(gi * (16 * _D), 8)
            idxg = plsc.load_gather(idx_v[ibuf], [tvec >> 7, tvec & (_W - 1)])
            for l in range(16):
                splat = lax.gather(
                    idxg, lanes[l][:, None], dnums, (1,),
                    mode=lax.GatherScatterMode.PROMISE_IN_BOUNDS)
                for k in range(_D // 16):
                    val = plsc.load_gather(tab_v, [splat, cols[k]])
                    out_v[ib][pl.ds(obase + l * _D + 16 * k, 16)] = val
            return tvec + 16

        lax.fori_loop(0, _H // 16, group, tvec0)
        pltpu.async_copy(out_v[ib], out_hbm.at[pl.ds(tok0 * _D, _H * _D)],
                         wsem[ib])

    fire_idx(0, 0)

    def pair(g, _):
        for b in range(2):
            i = 2 * g + b
            tok0 = tok_base + i * _T
            pltpu.make_async_copy(
                idx_hbm.at[pl.ds(row_base + i * _RCH, _RCH), :],
                idx_v[b], isem[b]).wait()

            @pl.when(i + 1 < _CHUNKS)
            def _():
                fire_idx(i + 1, 1 - b)

            for hb in range(2):
                @pl.when(i >= 1)
                def _():
                    pltpu.make_async_copy(
                        out_v[hb], out_hbm.at[pl.ds(tok0 * _D, _H * _D)],
                        wsem[hb]).wait()
                half(hb, b, iota + hb * _H, tok0 + hb * _H)
        return ()

    lax.fori_loop(0, _CHUNKS // 2, pair, ())
    for b in range(2):
        pltpu.make_async_copy(out_v[b],
                              out_hbm.at[pl.ds(tok_base * _D, _H * _D)],
                              wsem[b]).wait()


def kernel(inputs, table):
    idx = inputs.astype(jnp.int32).reshape(_NR, _W)
    out = pl.kernel(
        _sc_body,
        out_type=jax.ShapeDtypeStruct((_N * _D,), jnp.float32),
        mesh=plsc.VectorSubcoreMesh(core_axis_name="c", subcore_axis_name="s"),
        compiler_params=pltpu.CompilerParams(needs_layout_passes=False),
        scratch_types=[
            pltpu.VMEM((7, _D), jnp.float32),
            pltpu.VMEM((_RCH, _W), jnp.int32),
            pltpu.VMEM((_RCH, _W), jnp.int32),
            pltpu.VMEM((_H * _D,), jnp.float32),
            pltpu.VMEM((_H * _D,), jnp.float32),
            pltpu.SemaphoreType.DMA,
            pltpu.SemaphoreType.DMA,
            pltpu.SemaphoreType.DMA,
            pltpu.SemaphoreType.DMA,
        ],
    )(idx, table)
    return out.reshape(_B, _S, _D)
